# baseline (speedup 1.0000x reference)
import functools

import jax
import jax.numpy as jnp
from jax import lax
from jax.experimental import pallas as pl
from jax.experimental.pallas import tpu as pltpu
from jax.experimental.pallas import tpu_sc as plsc

DP = 16
BLK = 4


@functools.lru_cache(maxsize=None)
def _make_gather(B: int, L: int, V: int, S: int, K: int):
    D = S * K
    try:
        info = plsc.get_sparse_core_info()
        NC, NS = info.num_cores, info.num_subcores
    except Exception:
        NC, NS = 2, 16
    NW = NC * NS
    CB = B // 128
    CPW = CB // NW
    assert CB % NW == 0 and L % BLK == 0 and B % 128 == 0
    NBLK = L // BLK
    assert NBLK % 2 == 0
    CHTOK = 128 * L

    mesh = plsc.VectorSubcoreMesh(core_axis_name="c", subcore_axis_name="s",
                                  num_cores=NC, num_subcores=NS)

    @functools.partial(
        pl.kernel,
        out_type=[
            jax.ShapeDtypeStruct((S * L, K * B), jnp.float32),
            jax.ShapeDtypeStruct((S * L, K * B), jnp.float32),
        ],
        mesh=mesh,
        scratch_types=[
            pltpu.VMEM((CHTOK,), jnp.int32),
            pltpu.VMEM((L, 128), jnp.int32),
            pltpu.VMEM((2, BLK * 128, DP), jnp.float32),
            pltpu.VMEM((2, BLK * 128, DP), jnp.float32),
            pltpu.VMEM((2, BLK * S, K * 128), jnp.float32),
            pltpu.VMEM((2, BLK * S, K * 128), jnp.float32),
            pltpu.SemaphoreType.DMA,
            pltpu.SemaphoreType.DMA,
            pltpu.SemaphoreType.DMA,
            pltpu.SemaphoreType.DMA,
            pltpu.SemaphoreType.DMA,
            pltpu.SemaphoreType.DMA,
            pltpu.SemaphoreType.DMA,
            pltpu.SemaphoreType.DMA,
        ],
        compiler_params=pltpu.CompilerParams(use_tc_tiling_on_sc=False,
                                             needs_layout_passes=False),
    )
    def sc_gather(ids_hbm, amp_hbm, phase_hbm, oa, op,
                  raw_v, idsT_v, arows, prows, astg, pstg,
                  sga0, sga1, sgp0, sgp1, soa0, soa1, sop0, sop1):
        wid = lax.axis_index("s") * NC + lax.axis_index("c")
        sga = (sga0, sga1)
        sgp = (sgp0, sgp1)
        soa = (soa0, soa1)
        sop = (sop0, sop1)
        lane = lax.iota(jnp.int32, 16)
        lane_l = lane * L

        def fire_g(slot, blk):
            for i in range(BLK):
                cell = blk * BLK + i
                pltpu.async_copy(amp_hbm.at[idsT_v.at[cell]],
                                 arows.at[slot, pl.ds(i * 128, 128)], sga[slot])
                pltpu.async_copy(phase_hbm.at[idsT_v.at[cell]],
                                 prows.at[slot, pl.ds(i * 128, 128)], sgp[slot])

        def wait_g(slot):
            pltpu.make_async_copy(amp_hbm.at[pl.ds(0, BLK * 128)],
                                  arows.at[slot], sga[slot]).wait()
            pltpu.make_async_copy(phase_hbm.at[pl.ds(0, BLK * 128)],
                                  prows.at[slot], sgp[slot]).wait()

        def wait_out(slot):
            pltpu.make_async_copy(astg.at[slot],
                                  oa.at[pl.ds(0, BLK * S), pl.ds(0, K * 128)],
                                  soa[slot]).wait()
            pltpu.make_async_copy(pstg.at[slot],
                                  op.at[pl.ds(0, BLK * S), pl.ds(0, K * 128)],
                                  sop[slot]).wait()

        def repack(slot):
            rs_a = arows.at[slot]
            rs_p = prows.at[slot]
            cvecs = [jnp.full((16,), c, jnp.int32) for c in range(S * K)]
            for i in range(BLK):
                for jjp in range(4):
                    jjs = (2 * jjp, 2 * jjp + 1)
                    ridxs = [lane + (i * 128 + 16 * jj) for jj in jjs]
                    vas = [[plsc.load_gather(rs_a, [ridx, cvecs[c]])
                            for c in range(S * K)] for ridx in ridxs]
                    vps = [[plsc.load_gather(rs_p, [ridx, cvecs[c]])
                            for c in range(S * K)] for ridx in ridxs]
                    for h, jj in enumerate(jjs):
                        for s in range(S):
                            for k in range(K):
                                c = s * K + k
                                astg[slot, i * S + s,
                                     pl.ds(k * 128 + 16 * jj, 16)] = vas[h][c]
                                pstg[slot, i * S + s,
                                     pl.ds(k * 128 + 16 * jj, 16)] = vps[h][c]

        def fire_out(slot, blk, bc):
            pltpu.async_copy(
                astg.at[slot],
                oa.at[pl.ds(blk * BLK * S, BLK * S), pl.ds(bc * K * 128, K * 128)],
                soa[slot])
            pltpu.async_copy(
                pstg.at[slot],
                op.at[pl.ds(blk * BLK * S, BLK * S), pl.ds(bc * K * 128, K * 128)],
                sop[slot])

        def chunk_body(chunk, _):
            bc = wid * CPW + chunk
            pltpu.sync_copy(ids_hbm.at[pl.ds(bc * CHTOK, CHTOK)], raw_v)

            def tbody(l, _):
                for jj in range(8):
                    ridx = lane_l + (16 * jj * L) + l
                    v = plsc.load_gather(raw_v, [ridx])
                    idsT_v[l, pl.ds(16 * jj, 16)] = v
                return ()

            lax.fori_loop(0, L, tbody, ())

            fire_g(0, 0)
            fire_g(1, 1)

            def pbody(bp, _):
                for slot in range(2):
                    blk = 2 * bp + slot
                    wait_g(slot)

                    @pl.when(bp > 0)
                    def _():
                        wait_out(slot)

                    repack(slot)
                    fire_out(slot, blk, bc)
                    nxt = blk + 2

                    @pl.when(nxt < NBLK)
                    def _():
                        fire_g(slot, nxt)

                return ()

            lax.fori_loop(0, NBLK // 2, pbody, ())
            wait_out(0)
            wait_out(1)
            return ()

        lax.fori_loop(0, CPW, chunk_body, ())

    return sc_gather


def kernel(ids, delta_amp, delta_phase):
    B, L = ids.shape
    V, S, K = delta_amp.shape
    D = S * K
    ids_f = ids.reshape(B * L)
    amp16 = jnp.pad(delta_amp.reshape(V, D), ((0, 0), (0, DP - D)))
    phase16 = jnp.pad(delta_phase.reshape(V, D), ((0, 0), (0, DP - D)))
    oa, op = _make_gather(B, L, V, S, K)(ids_f, amp16, phase16)

    def to_native(o):
        z = o.reshape(L, S, B // 128, K, 128)
        return z.transpose(2, 4, 0, 1, 3).reshape(B, L, S, K)

    return to_native(oa), to_native(op)


# speedup vs baseline: 1.3563x; 1.3563x over previous
import functools

import jax
import jax.numpy as jnp
from jax import lax
from jax.experimental import pallas as pl
from jax.experimental.pallas import tpu as pltpu
from jax.experimental.pallas import tpu_sc as plsc

DP = 16
BLK = 4


@functools.lru_cache(maxsize=None)
def _make_gather(B: int, L: int, V: int, S: int, K: int):
    D = S * K
    try:
        info = plsc.get_sparse_core_info()
        NC, NS = info.num_cores, info.num_subcores
    except Exception:
        NC, NS = 2, 16
    NW = NC * NS
    CB = B // 128
    CPW = CB // NW
    assert CB % NW == 0 and L % BLK == 0 and B % 128 == 0
    NBLK = L // BLK
    assert NBLK % 2 == 0
    CHTOK = 128 * L

    mesh = plsc.VectorSubcoreMesh(core_axis_name="c", subcore_axis_name="s",
                                  num_cores=NC, num_subcores=NS)

    @functools.partial(
        pl.kernel,
        out_type=[
            jax.ShapeDtypeStruct((S * L, K * B), jnp.float32),
            jax.ShapeDtypeStruct((S * L, K * B), jnp.float32),
        ],
        mesh=mesh,
        scratch_types=[
            pltpu.VMEM((CHTOK,), jnp.int32),
            pltpu.VMEM((L, 128), jnp.int32),
            pltpu.VMEM((2, BLK * 128, DP), jnp.float32),
            pltpu.VMEM((2, BLK * 128, DP), jnp.float32),
            pltpu.VMEM((2, BLK * S, K * 128), jnp.float32),
            pltpu.VMEM((2, BLK * S, K * 128), jnp.float32),
            pltpu.SemaphoreType.DMA,
            pltpu.SemaphoreType.DMA,
            pltpu.SemaphoreType.DMA,
            pltpu.SemaphoreType.DMA,
            pltpu.SemaphoreType.DMA,
            pltpu.SemaphoreType.DMA,
            pltpu.SemaphoreType.DMA,
            pltpu.SemaphoreType.DMA,
        ],
        compiler_params=pltpu.CompilerParams(use_tc_tiling_on_sc=False,
                                             needs_layout_passes=False),
    )
    def sc_gather(ids_hbm, amp_hbm, phase_hbm, oa, op,
                  raw_v, idsT_v, arows, prows, astg, pstg,
                  sga0, sga1, sgp0, sgp1, soa0, soa1, sop0, sop1):
        wid = lax.axis_index("s") * NC + lax.axis_index("c")
        sga = (sga0, sga1)
        sgp = (sgp0, sgp1)
        soa = (soa0, soa1)
        sop = (sop0, sop1)
        lane = lax.iota(jnp.int32, 16)
        lane_l = lane * L

        def fire_g(slot, blk):
            for i in range(BLK):
                cell = blk * BLK + i
                pltpu.async_copy(amp_hbm.at[idsT_v.at[cell]],
                                 arows.at[slot, pl.ds(i * 128, 128)], sga[slot])
                pltpu.async_copy(phase_hbm.at[idsT_v.at[cell]],
                                 prows.at[slot, pl.ds(i * 128, 128)], sgp[slot])

        def wait_g(slot):
            pltpu.make_async_copy(amp_hbm.at[pl.ds(0, BLK * 128)],
                                  arows.at[slot], sga[slot]).wait()
            pltpu.make_async_copy(phase_hbm.at[pl.ds(0, BLK * 128)],
                                  prows.at[slot], sgp[slot]).wait()

        def wait_out(slot):
            pltpu.make_async_copy(astg.at[slot],
                                  oa.at[pl.ds(0, BLK * S), pl.ds(0, K * 128)],
                                  soa[slot]).wait()
            pltpu.make_async_copy(pstg.at[slot],
                                  op.at[pl.ds(0, BLK * S), pl.ds(0, K * 128)],
                                  sop[slot]).wait()

        def repack(slot):
            rs_a = arows.at[slot]
            rs_p = prows.at[slot]
            cvecs = [jnp.full((16,), c, jnp.int32) for c in range(S * K)]
            for i in range(BLK):
                for jj in range(8):
                    ridx = lane + (i * 128 + 16 * jj)
                    vas = [plsc.load_gather(rs_a, [ridx, cvecs[c]])
                           for c in range(S * K)]
                    vps = [plsc.load_gather(rs_p, [ridx, cvecs[c]])
                           for c in range(S * K)]
                    for s in range(S):
                        for k in range(K):
                            c = s * K + k
                            astg[slot, i * S + s,
                                 pl.ds(k * 128 + 16 * jj, 16)] = vas[c]
                            pstg[slot, i * S + s,
                                 pl.ds(k * 128 + 16 * jj, 16)] = vps[c]

        def fire_out(slot, blk, bc):
            pltpu.async_copy(
                astg.at[slot],
                oa.at[pl.ds(blk * BLK * S, BLK * S), pl.ds(bc * K * 128, K * 128)],
                soa[slot])
            pltpu.async_copy(
                pstg.at[slot],
                op.at[pl.ds(blk * BLK * S, BLK * S), pl.ds(bc * K * 128, K * 128)],
                sop[slot])

        def chunk_body(chunk, _):
            bc = wid * CPW + chunk
            pltpu.sync_copy(ids_hbm.at[pl.ds(bc * CHTOK, CHTOK)], raw_v)

            def tbody(l, _):
                for jj in range(8):
                    ridx = lane_l + (16 * jj * L) + l
                    v = plsc.load_gather(raw_v, [ridx])
                    idsT_v[l, pl.ds(16 * jj, 16)] = v
                return ()

            lax.fori_loop(0, L, tbody, ())

            fire_g(0, 0)
            fire_g(1, 1)

            def pbody(bp, _):
                for slot in range(2):
                    blk = 2 * bp + slot
                    wait_g(slot)

                    @pl.when(bp > 0)
                    def _():
                        wait_out(slot)

                    repack(slot)
                    fire_out(slot, blk, bc)
                    nxt = blk + 2

                    @pl.when(nxt < NBLK)
                    def _():
                        fire_g(slot, nxt)

                return ()

            lax.fori_loop(0, NBLK // 2, pbody, ())
            wait_out(0)
            wait_out(1)
            return ()

        lax.fori_loop(0, CPW, chunk_body, ())

    return sc_gather


def kernel(ids, delta_amp, delta_phase):
    B, L = ids.shape
    V, S, K = delta_amp.shape
    D = S * K
    ids_f = ids.reshape(B * L)
    amp16 = jnp.pad(delta_amp, ((0, 0), (0, 1), (0, 0))).reshape(V, DP)
    phase16 = jnp.pad(delta_phase, ((0, 0), (0, 1), (0, 0))).reshape(V, DP)
    oa, op = _make_gather(B, L, V, S, K)(ids_f, amp16, phase16)

    def to_native(o):
        z = o.reshape(L, S, B // 128, K, 128)
        return z.transpose(2, 4, 0, 1, 3).reshape(B, L, S, K)

    return to_native(oa), to_native(op)
